# baseline (speedup 1.0000x reference)
import jax
import jax.numpy as jnp
from jax.experimental import pallas as pl

N = 10000
G = 64
HID = 128
NUM_CLASSES = 40


def _fc_body(pooled_ref, w_ref, b_ref, out_ref):
    out_ref[...] = pooled_ref[...] @ w_ref[...] + b_ref[...]


def _gat_conv(x, src, dst, W, a_s, a_d, b, H, C):
    n = x.shape[0]
    xp = (x @ W).reshape(n, H, C)
    alpha_s = jnp.sum(xp * a_s[None, :, :], axis=-1)
    alpha_d = jnp.sum(xp * a_d[None, :, :], axis=-1)
    e = jax.nn.leaky_relu(alpha_s[src] + alpha_d[dst], 0.2)
    emax = jax.ops.segment_max(e, dst, num_segments=n)
    emax = jnp.where(jnp.isfinite(emax), emax, 0.0)
    ee = jnp.exp(e - emax[dst])
    denom = jax.ops.segment_sum(ee, dst, num_segments=n)
    alpha = ee / (denom[dst] + 1e-16)
    out = jax.ops.segment_sum(xp[src] * alpha[:, :, None], dst, num_segments=n)
    return out.reshape(n, H * C) + b


def kernel(x, edge_index, batch, W1, a1_src, a1_dst, b1, W2, a2_src, a2_dst, b2, Wfc, bfc):
    src, dst = edge_index[0], edge_index[1]
    h = jax.nn.elu(_gat_conv(x, src, dst, W1, a1_src, a1_dst, b1, 8, HID))
    h = jax.nn.elu(_gat_conv(h, src, dst, W2, a2_src, a2_dst, b2, 1, HID))
    sums = jax.ops.segment_sum(h, batch, num_segments=G)
    counts = jax.ops.segment_sum(jnp.ones((h.shape[0], 1), h.dtype), batch, num_segments=G)
    pooled = sums / jnp.maximum(counts, 1.0)
    return pl.pallas_call(
        _fc_body,
        out_shape=jax.ShapeDtypeStruct((G, NUM_CLASSES), jnp.float32),
    )(pooled, Wfc, bfc[None, :])


# speedup vs baseline: 14.6073x; 14.6073x over previous
import jax
import jax.numpy as jnp
from jax import lax
from jax.experimental import pallas as pl
from jax.experimental.pallas import tpu as pltpu
from jax.experimental.pallas import tpu_sc as plsc

N = 10000
E = 320000
F = 128
H1 = 8
C = 128
G = 64
NCLS = 40

ROWS = 10240
ADP = ROWS + 8
STRIPE = E // 16
EBLK = 2000


def _sc_geom(H):
    if H == 8:
        return dict(CHUNK=256, NCHUNK=40, KB=32, ZP=48)
    return dict(CHUNK=1024, NCHUNK=10, KB=96, ZP=64)


def _t1_body(x_ref, w1_ref, as_ref, ad_ref, oas_ref, oad_ref):
    nb = x_ref.shape[0]
    w1 = w1_ref[...]
    avs = as_ref[...]
    avd = ad_ref[...]
    vs = jnp.concatenate(
        [jnp.sum(w1[:, h * C:(h + 1) * C] * avs[h:h + 1, :], axis=1,
                 keepdims=True) for h in range(H1)], axis=1)
    vd = jnp.concatenate(
        [jnp.sum(w1[:, h * C:(h + 1) * C] * avd[h:h + 1, :], axis=1,
                 keepdims=True) for h in range(H1)], axis=1)
    xb = x_ref[...]
    a = jnp.dot(xb, vs, preferred_element_type=jnp.float32)
    b = jnp.dot(xb, vd, preferred_element_type=jnp.float32)
    z = jnp.zeros((nb, 112), jnp.float32)
    oas_ref[...] = jnp.concatenate([a, a, z], axis=1)
    oad_ref[...] = jnp.concatenate([b, b, z], axis=1)


def _t1(x, W1, a1_src, a1_dst):
    nb = 10
    return pl.pallas_call(
        _t1_body,
        grid=(nb,),
        in_specs=[
            pl.BlockSpec((N // nb, F), lambda i: (i, 0)),
            pl.BlockSpec((F, H1 * C), lambda i: (0, 0)),
            pl.BlockSpec((H1, C), lambda i: (0, 0)),
            pl.BlockSpec((H1, C), lambda i: (0, 0)),
        ],
        out_specs=[
            pl.BlockSpec((N // nb, 128), lambda i: (i, 0)),
            pl.BlockSpec((N // nb, 128), lambda i: (i, 0)),
        ],
        out_shape=[
            jax.ShapeDtypeStruct((N, 128), jnp.float32),
            jax.ShapeDtypeStruct((N, 128), jnp.float32),
        ],
    )(x, W1, a1_src, a1_dst)


def _make_sc_body(H):
    SR = H + 1
    g_ = _sc_geom(H)
    CHUNK, NCHUNK, KB, ZP = g_["CHUNK"], g_["NCHUNK"], g_["KB"], g_["ZP"]

    def body(src_r, dst_r, x_r, as_r, ad_r,
             agg_r,
             acc_sh,
             src_blk, dst_blk, src_sel, dst_sel,
             xbuf, asbuf, adbuf, outbuf, idxbuf, dstloc_b, zbuf,
             sem1, sem2, sem3, sem4):
        c = lax.axis_index("c")
        s = lax.axis_index("s")
        estripe = s * STRIPE
        rows_t = (CHUNK // 16) * SR

        def zb_body(e, _):
            for j in range(8):
                zbuf[e, pl.ds(16 * j, 16)] = jnp.zeros((16,), jnp.float32)
            return 0

        lax.fori_loop(0, ZP, zb_body, 0)

        for ci in range(NCHUNK // 2):
            chunk = 2 * ci + c
            lo = chunk * CHUNK

            for i in range(rows_t // ZP):
                pltpu.sync_copy(
                    zbuf, acc_sh.at[pl.ds(rows_t * s + ZP * i, ZP)])
            plsc.subcore_barrier()

            def blk_loop(blk, k):
                pltpu.sync_copy(src_r.at[pl.ds(estripe + EBLK * blk, EBLK)],
                                src_blk)
                pltpu.sync_copy(dst_r.at[pl.ds(estripe + EBLK * blk, EBLK)],
                                dst_blk)

                def g_body(g, k):
                    dv = dst_blk[pl.ds(g * 16, 16)]
                    sv = src_blk[pl.ds(g * 16, 16)]
                    m = (dv >= lo) & (dv < lo + CHUNK)
                    cum = plsc.cumsum(m.astype(jnp.int32))
                    pos = k + cum - 1
                    plsc.store_scatter(src_sel, [pos], sv, mask=m)
                    plsc.store_scatter(dst_sel, [pos], dv, mask=m)
                    return k + cum[15]

                return lax.fori_loop(0, EBLK // 16, g_body, k)

            k = lax.fori_loop(0, STRIPE // EBLK, blk_loop, jnp.int32(0))

            pad_dst = lo + CHUNK + (s % 8)
            for i in range(KB // 16):
                src_sel[pl.ds(k + 16 * i, 16)] = jnp.zeros((16,), jnp.int32)
                dst_sel[pl.ds(k + 16 * i, 16)] = jnp.full((16,), pad_dst,
                                                          jnp.int32)

            nb = (k + (KB - 1)) // KB

            def batch_body(b, _):
                off = b * KB
                for i in range(KB // 16):
                    dstloc_b[pl.ds(16 * i, 16)] = (
                        dst_sel[pl.ds(off + 16 * i, 16)] - lo)
                iot = lax.iota(jnp.int32, 16)
                for g in range(KB * SR // 16):
                    fl = iot + 16 * g
                    ev = lax.div(fl, jnp.int32(SR))
                    rv = fl - ev * SR
                    dl = plsc.load_gather(dstloc_b, [ev])
                    idxbuf[pl.ds(16 * g, 16)] = dl * SR + rv
                cp1 = pltpu.async_copy(
                    x_r.at[src_sel.at[pl.ds(off, KB)]], xbuf, sem1)
                cp2 = pltpu.async_copy(
                    as_r.at[src_sel.at[pl.ds(off, KB)]], asbuf, sem2)
                cp3 = pltpu.async_copy(
                    ad_r.at[dst_sel.at[pl.ds(off, KB)]], adbuf, sem3)
                cp1.wait()
                cp2.wait()
                cp3.wait()

                def e_body(e, _):
                    eb = e * SR
                    av = asbuf[e, pl.ds(0, 16)]
                    bv = adbuf[e, pl.ds(0, 16)]
                    sm = av + bv
                    sm = jnp.where(sm >= 0.0, sm, 0.2 * sm)
                    eev = jnp.exp(sm)
                    outbuf[eb + H, pl.ds(0, 16)] = eev
                    xr = [xbuf[e, pl.ds(16 * j, 16)] for j in range(8)]
                    for h in range(H):
                        vv = jnp.full((16,), eev[h], jnp.float32)
                        for j in range(8):
                            outbuf[eb + h, pl.ds(16 * j, 16)] = xr[j] * vv
                    return 0

                lax.fori_loop(0, KB, e_body, 0)

                pltpu.sync_copy(outbuf, acc_sh.at[idxbuf], add=True)
                return 0

            lax.fori_loop(0, nb, batch_body, 0)
            plsc.subcore_barrier()

            for i in range(rows_t // ZP):
                pltpu.sync_copy(
                    acc_sh.at[pl.ds(rows_t * s + ZP * i, ZP)],
                    agg_r.at[pl.ds(lo * SR + rows_t * s + ZP * i, ZP)])
            plsc.subcore_barrier()

    return body


def _sc_agg(src, dst, feat, asrc, adstp, H):
    SR = H + 1
    g_ = _sc_geom(H)
    CHUNK, KB, ZP = g_["CHUNK"], g_["KB"], g_["ZP"]
    mesh = plsc.VectorSubcoreMesh(core_axis_name="c", subcore_axis_name="s")
    f = pl.kernel(
        _make_sc_body(H),
        out_type=jax.ShapeDtypeStruct((ROWS * SR, 128), jnp.float32),
        mesh=mesh,
        compiler_params=pltpu.CompilerParams(needs_layout_passes=False),
        scratch_types=[
            pltpu.VMEM_SHARED(((CHUNK + 8) * SR, 128), jnp.float32),
            pltpu.VMEM((EBLK,), jnp.int32),
            pltpu.VMEM((EBLK,), jnp.int32),
            pltpu.VMEM((STRIPE + KB,), jnp.int32),
            pltpu.VMEM((STRIPE + KB,), jnp.int32),
            pltpu.VMEM((KB, F), jnp.float32),
            pltpu.VMEM((KB, 128), jnp.float32),
            pltpu.VMEM((KB, 128), jnp.float32),
            pltpu.VMEM((KB * SR, 128), jnp.float32),
            pltpu.VMEM((KB * SR,), jnp.int32),
            pltpu.VMEM((KB,), jnp.int32),
            pltpu.VMEM((ZP, 128), jnp.float32),
            pltpu.SemaphoreType.DMA,
            pltpu.SemaphoreType.DMA,
            pltpu.SemaphoreType.DMA,
            pltpu.SemaphoreType.DMA,
        ],
    )
    return f(src, dst, feat, asrc, adstp)


def _t2_body(agg_ref, w1_ref, b1_ref, w2_ref, a2s_ref, a2d_ref,
             xp2_ref, as2_ref, ad2_ref):
    nb = agg_ref.shape[0]
    dr = 1.0 / (agg_ref[:, H1 * C:H1 * C + H1] + 1e-16)
    xp2 = jnp.zeros((nb, C), jnp.float32)
    for h in range(H1):
        ag = agg_ref[:, h * C:(h + 1) * C] * dr[:, h:h + 1]
        y = jnp.dot(ag, w1_ref[:, h * C:(h + 1) * C],
                    preferred_element_type=jnp.float32)
        y = y + b1_ref[0, h * C:(h + 1) * C][None, :]
        y = jnp.where(y > 0.0, y, jnp.exp(y) - 1.0)
        xp2 = xp2 + jnp.dot(y, w2_ref[h * C:(h + 1) * C, :],
                            preferred_element_type=jnp.float32)
    xp2_ref[...] = xp2
    as2 = jnp.sum(xp2 * a2s_ref[...], axis=1, keepdims=True)
    ad2 = jnp.sum(xp2 * a2d_ref[...], axis=1, keepdims=True)
    as2_ref[...] = jnp.broadcast_to(as2, (nb, 128))
    ad2_ref[...] = jnp.broadcast_to(ad2, (nb, 128))


def _t2(agg1, W1, b1, W2, a2_src, a2_dst):
    nb = 10
    blk = ROWS // nb
    return pl.pallas_call(
        _t2_body,
        grid=(nb,),
        in_specs=[
            pl.BlockSpec((blk, H1 * C + 128), lambda i: (i, 0)),
            pl.BlockSpec((F, H1 * C), lambda i: (0, 0)),
            pl.BlockSpec((1, H1 * C), lambda i: (0, 0)),
            pl.BlockSpec((H1 * C, C), lambda i: (0, 0)),
            pl.BlockSpec((1, C), lambda i: (0, 0)),
            pl.BlockSpec((1, C), lambda i: (0, 0)),
        ],
        out_specs=[
            pl.BlockSpec((blk, C), lambda i: (i, 0)),
            pl.BlockSpec((blk, 128), lambda i: (i, 0)),
            pl.BlockSpec((blk, 128), lambda i: (i, 0)),
        ],
        out_shape=[
            jax.ShapeDtypeStruct((ROWS, C), jnp.float32),
            jax.ShapeDtypeStruct((ROWS, 128), jnp.float32),
            jax.ShapeDtypeStruct((ROWS, 128), jnp.float32),
        ],
    )(agg1, W1, b1[None, :], W2, a2_src, a2_dst)


def _t3_body(p_ref, b2_ref, batch_ref, wfc_ref, bfc_ref, out_ref,
             sums_sc, cnts_sc):
    i = pl.program_id(0)
    d = p_ref[:, C:C + 1] + 1e-16
    h2 = p_ref[:, 0:C] / d + b2_ref[...]
    h2 = jnp.where(h2 > 0.0, h2, jnp.exp(h2) - 1.0)
    bb = batch_ref[0, 0, :][None, :]
    gid = lax.broadcasted_iota(jnp.int32, (G, 1), 0)
    mask = (bb == gid).astype(jnp.float32)

    @pl.when(i == 0)
    def _():
        sums_sc[...] = jnp.zeros_like(sums_sc)
        cnts_sc[...] = jnp.zeros_like(cnts_sc)

    sums_sc[...] += jnp.dot(mask, h2, preferred_element_type=jnp.float32)
    cnts_sc[...] += jnp.sum(mask, axis=1, keepdims=True)

    @pl.when(i == pl.num_programs(0) - 1)
    def _():
        pooled = sums_sc[...] / jnp.maximum(cnts_sc[...], 1.0)
        out_ref[...] = jnp.dot(pooled, wfc_ref[...],
                               preferred_element_type=jnp.float32) + bfc_ref[...]


def _t3(agg2, b2, batch, Wfc, bfc):
    nb = 10
    blk = N // nb
    return pl.pallas_call(
        _t3_body,
        grid=(nb,),
        in_specs=[
            pl.BlockSpec((blk, 256), lambda i: (i, 0)),
            pl.BlockSpec((1, C), lambda i: (0, 0)),
            pl.BlockSpec((1, 1, blk), lambda i: (i, 0, 0)),
            pl.BlockSpec((C, NCLS), lambda i: (0, 0)),
            pl.BlockSpec((1, NCLS), lambda i: (0, 0)),
        ],
        out_specs=pl.BlockSpec((G, NCLS), lambda i: (0, 0)),
        out_shape=jax.ShapeDtypeStruct((G, NCLS), jnp.float32),
        scratch_shapes=[
            pltpu.VMEM((G, C), jnp.float32),
            pltpu.VMEM((G, 1), jnp.float32),
        ],
    )(agg2, b2[None, :], batch.reshape(nb, 1, blk), Wfc, bfc[None, :])


def kernel(x, edge_index, batch, W1, a1_src, a1_dst, b1, W2, a2_src, a2_dst,
           b2, Wfc, bfc):
    src = edge_index[0]
    dst = edge_index[1]

    as1, ad1 = _t1(x, W1, a1_src, a1_dst)
    ad1p = jnp.concatenate(
        [ad1, jnp.zeros((ADP - N, 128), jnp.float32)], axis=0)

    agg1 = _sc_agg(src, dst, x, as1, ad1p, H1).reshape(ROWS, (H1 + 1) * 128)

    xp2, as2, ad2 = _t2(agg1, W1, b1, W2, a2_src, a2_dst)
    ad2p = jnp.concatenate(
        [ad2, jnp.zeros((ADP - ROWS, 128), jnp.float32)], axis=0)

    agg2 = _sc_agg(src, dst, xp2, as2, ad2p, 1).reshape(ROWS, 256)

    return _t3(agg2, b2, batch, Wfc, bfc)


# speedup vs baseline: 18.4773x; 1.2649x over previous
import jax
import jax.numpy as jnp
from jax import lax
from jax.experimental import pallas as pl
from jax.experimental.pallas import tpu as pltpu
from jax.experimental.pallas import tpu_sc as plsc

N = 10000
E = 320000
F = 128
H1 = 8
C = 128
G = 64
NCLS = 40

ROWS = 10240
ADP = ROWS + 8
STRIPE = E // 16
EBLK = 2000


def _sc_geom(H):
    if H == 8:
        return dict(CHUNK=256, NCHUNK=40, KB=16, ZP=48)
    return dict(CHUNK=1024, NCHUNK=10, KB=16, ZP=64)


def _t1_body(x_ref, w1_ref, as_ref, ad_ref, oas_ref, oad_ref):
    nb = x_ref.shape[0]
    w1 = w1_ref[...]
    avs = as_ref[...]
    avd = ad_ref[...]
    vs = jnp.concatenate(
        [jnp.sum(w1[:, h * C:(h + 1) * C] * avs[h:h + 1, :], axis=1,
                 keepdims=True) for h in range(H1)], axis=1)
    vd = jnp.concatenate(
        [jnp.sum(w1[:, h * C:(h + 1) * C] * avd[h:h + 1, :], axis=1,
                 keepdims=True) for h in range(H1)], axis=1)
    xb = x_ref[...]
    a = jnp.dot(xb, vs, preferred_element_type=jnp.float32)
    b = jnp.dot(xb, vd, preferred_element_type=jnp.float32)
    z = jnp.zeros((nb, 112), jnp.float32)
    oas_ref[...] = jnp.concatenate([a, a, z], axis=1)
    oad_ref[...] = jnp.concatenate([b, b, z], axis=1)


def _t1(x, W1, a1_src, a1_dst):
    nb = 10
    return pl.pallas_call(
        _t1_body,
        grid=(nb,),
        in_specs=[
            pl.BlockSpec((N // nb, F), lambda i: (i, 0)),
            pl.BlockSpec((F, H1 * C), lambda i: (0, 0)),
            pl.BlockSpec((H1, C), lambda i: (0, 0)),
            pl.BlockSpec((H1, C), lambda i: (0, 0)),
        ],
        out_specs=[
            pl.BlockSpec((N // nb, 128), lambda i: (i, 0)),
            pl.BlockSpec((N // nb, 128), lambda i: (i, 0)),
        ],
        out_shape=[
            jax.ShapeDtypeStruct((N, 128), jnp.float32),
            jax.ShapeDtypeStruct((N, 128), jnp.float32),
        ],
    )(x, W1, a1_src, a1_dst)


def _make_sc_body(H):
    SR = H + 1
    g_ = _sc_geom(H)
    CHUNK, NCHUNK, KB, ZP = g_["CHUNK"], g_["NCHUNK"], g_["KB"], g_["ZP"]

    def body(src_r, dst_r, x_r, as_r, ad_r,
             agg_r,
             acc_sh,
             src_blk, dst_blk, src_sel, dst_sel,
             xbuf, asbuf, adbuf, outbuf, idxbuf, dstloc_b, zbuf,
             gsem, ssem):
        c = lax.axis_index("c")
        s = lax.axis_index("s")
        estripe = s * STRIPE
        rows_t = (CHUNK // 16) * SR

        def zb_body(e, _):
            for j in range(8):
                zbuf[e, pl.ds(16 * j, 16)] = jnp.zeros((16,), jnp.float32)
            return 0

        lax.fori_loop(0, ZP, zb_body, 0)

        def chunk_body(ci, _c):
            chunk = 2 * ci + c
            lo = chunk * CHUNK

            for i in range(rows_t // ZP):
                pltpu.sync_copy(
                    zbuf, acc_sh.at[pl.ds(rows_t * s + ZP * i, ZP)])
            plsc.subcore_barrier()

            def blk_loop(blk, k):
                pltpu.sync_copy(src_r.at[pl.ds(estripe + EBLK * blk, EBLK)],
                                src_blk)
                pltpu.sync_copy(dst_r.at[pl.ds(estripe + EBLK * blk, EBLK)],
                                dst_blk)

                def g_body(g, k):
                    dv = dst_blk[pl.ds(g * 16, 16)]
                    sv = src_blk[pl.ds(g * 16, 16)]
                    m = (dv >= lo) & (dv < lo + CHUNK)
                    cum = plsc.cumsum(m.astype(jnp.int32))
                    pos = k + cum - 1
                    plsc.store_scatter(src_sel, [pos], sv, mask=m)
                    plsc.store_scatter(dst_sel, [pos], dv, mask=m)
                    return k + cum[15]

                return lax.fori_loop(0, EBLK // 16, g_body, k)

            k = lax.fori_loop(0, STRIPE // EBLK, blk_loop, jnp.int32(0))

            pad_dst = lo + CHUNK + (s % 8)
            for i in range(KB // 16):
                src_sel[pl.ds(k + 16 * i, 16)] = jnp.zeros((16,), jnp.int32)
                dst_sel[pl.ds(k + 16 * i, 16)] = jnp.full((16,), pad_dst,
                                                          jnp.int32)

            nb = (k + (KB - 1)) // KB

            def build_idx(bq, dq):
                offq = bq * KB
                for i in range(KB // 16):
                    dstloc_b[dq][pl.ds(16 * i, 16)] = (
                        dst_sel[pl.ds(offq + 16 * i, 16)] - lo)
                iot = lax.iota(jnp.int32, 16)
                for g in range(KB * SR // 16):
                    fl = iot + 16 * g
                    ev = lax.div(fl, jnp.int32(SR))
                    rv = fl - ev * SR
                    dl = plsc.load_gather(dstloc_b[dq], [ev])
                    idxbuf[dq][pl.ds(16 * g, 16)] = dl * SR + rv

            def issue_gathers(bq, dq):
                offq = bq * KB
                pltpu.async_copy(
                    x_r.at[src_sel.at[pl.ds(offq, KB)]], xbuf[dq], gsem[dq])
                pltpu.async_copy(
                    as_r.at[src_sel.at[pl.ds(offq, KB)]], asbuf[dq], gsem[dq])
                pltpu.async_copy(
                    ad_r.at[dst_sel.at[pl.ds(offq, KB)]], adbuf[dq], gsem[dq])

            def wait_gathers(bq, dq):
                offq = bq * KB
                pltpu.make_async_copy(
                    x_r.at[src_sel.at[pl.ds(offq, KB)]], xbuf[dq],
                    gsem[dq]).wait()
                pltpu.make_async_copy(
                    as_r.at[src_sel.at[pl.ds(offq, KB)]], asbuf[dq],
                    gsem[dq]).wait()
                pltpu.make_async_copy(
                    ad_r.at[dst_sel.at[pl.ds(offq, KB)]], adbuf[dq],
                    gsem[dq]).wait()

            def scatter_issue(dq):
                pltpu.async_copy(outbuf[dq], acc_sh.at[idxbuf[dq]],
                                 ssem[dq], add=True)

            def scatter_wait(dq):
                pltpu.make_async_copy(outbuf[dq], acc_sh.at[idxbuf[dq]],
                                      ssem[dq]).wait()

            @pl.when(nb > 0)
            def _():
                build_idx(0, 0)
                issue_gathers(0, 0)

            def run_batch(bb, d):
                wait_gathers(bb, d)

                @pl.when(bb + 1 < nb)
                def _():
                    @pl.when(bb >= 1)
                    def _():
                        scatter_wait(1 - d)

                    build_idx(bb + 1, 1 - d)
                    issue_gathers(bb + 1, 1 - d)

                def e_body(e, _):
                    eb = e * SR
                    av = asbuf[d][e, pl.ds(0, 16)]
                    bv = adbuf[d][e, pl.ds(0, 16)]
                    sm = av + bv
                    sm = jnp.where(sm >= 0.0, sm, 0.2 * sm)
                    eev = jnp.exp(sm)
                    outbuf[d][eb + H, pl.ds(0, 16)] = eev
                    xr = [xbuf[d][e, pl.ds(16 * j, 16)] for j in range(8)]
                    for h in range(H):
                        vv = jnp.full((16,), eev[h], jnp.float32)
                        for j in range(8):
                            outbuf[d][eb + h, pl.ds(16 * j, 16)] = xr[j] * vv
                    return 0

                lax.fori_loop(0, KB, e_body, 0)
                scatter_issue(d)

            def pair_body(p, _):
                for d in range(2):
                    bb = 2 * p + d

                    @pl.when(bb < nb)
                    def _(bb=bb, d=d):
                        run_batch(bb, d)
                return 0

            lax.fori_loop(0, (nb + 1) // 2, pair_body, 0)

            @pl.when(nb >= 2)
            def _():
                scatter_wait(0)
                scatter_wait(1)

            @pl.when(nb == 1)
            def _():
                scatter_wait(0)

            plsc.subcore_barrier()

            for i in range(rows_t // ZP):
                pltpu.sync_copy(
                    acc_sh.at[pl.ds(rows_t * s + ZP * i, ZP)],
                    agg_r.at[pl.ds(lo * SR + rows_t * s + ZP * i, ZP)])
            plsc.subcore_barrier()
            return 0

        lax.fori_loop(0, NCHUNK // 2, chunk_body, 0)

    return body


def _sc_agg(src, dst, feat, asrc, adstp, H):
    SR = H + 1
    g_ = _sc_geom(H)
    CHUNK, KB, ZP = g_["CHUNK"], g_["KB"], g_["ZP"]
    mesh = plsc.VectorSubcoreMesh(core_axis_name="c", subcore_axis_name="s")
    f = pl.kernel(
        _make_sc_body(H),
        out_type=jax.ShapeDtypeStruct((ROWS * SR, 128), jnp.float32),
        mesh=mesh,
        compiler_params=pltpu.CompilerParams(needs_layout_passes=False),
        scratch_types=[
            pltpu.VMEM_SHARED(((CHUNK + 8) * SR, 128), jnp.float32),
            pltpu.VMEM((EBLK,), jnp.int32),
            pltpu.VMEM((EBLK,), jnp.int32),
            pltpu.VMEM((STRIPE + KB,), jnp.int32),
            pltpu.VMEM((STRIPE + KB,), jnp.int32),
            tuple(pltpu.VMEM((KB, F), jnp.float32) for _ in range(2)),
            tuple(pltpu.VMEM((KB, 128), jnp.float32) for _ in range(2)),
            tuple(pltpu.VMEM((KB, 128), jnp.float32) for _ in range(2)),
            tuple(pltpu.VMEM((KB * SR, 128), jnp.float32) for _ in range(2)),
            tuple(pltpu.VMEM((KB * SR,), jnp.int32) for _ in range(2)),
            tuple(pltpu.VMEM((KB,), jnp.int32) for _ in range(2)),
            pltpu.VMEM((ZP, 128), jnp.float32),
            (pltpu.SemaphoreType.DMA, pltpu.SemaphoreType.DMA),
            (pltpu.SemaphoreType.DMA, pltpu.SemaphoreType.DMA),
        ],
    )
    return f(src, dst, feat, asrc, adstp)


def _t2_body(agg_ref, w1_ref, b1_ref, w2_ref, a2s_ref, a2d_ref,
             xp2_ref, as2_ref, ad2_ref):
    nb = agg_ref.shape[0]
    dr = 1.0 / (agg_ref[:, H1 * C:H1 * C + H1] + 1e-16)
    xp2 = jnp.zeros((nb, C), jnp.float32)
    for h in range(H1):
        ag = agg_ref[:, h * C:(h + 1) * C] * dr[:, h:h + 1]
        y = jnp.dot(ag, w1_ref[:, h * C:(h + 1) * C],
                    preferred_element_type=jnp.float32)
        y = y + b1_ref[0, h * C:(h + 1) * C][None, :]
        y = jnp.where(y > 0.0, y, jnp.exp(y) - 1.0)
        xp2 = xp2 + jnp.dot(y, w2_ref[h * C:(h + 1) * C, :],
                            preferred_element_type=jnp.float32)
    xp2_ref[...] = xp2
    as2 = jnp.sum(xp2 * a2s_ref[...], axis=1, keepdims=True)
    ad2 = jnp.sum(xp2 * a2d_ref[...], axis=1, keepdims=True)
    as2_ref[...] = jnp.broadcast_to(as2, (nb, 128))
    ad2_ref[...] = jnp.broadcast_to(ad2, (nb, 128))


def _t2(agg1, W1, b1, W2, a2_src, a2_dst):
    nb = 10
    blk = ROWS // nb
    return pl.pallas_call(
        _t2_body,
        grid=(nb,),
        in_specs=[
            pl.BlockSpec((blk, H1 * C + 128), lambda i: (i, 0)),
            pl.BlockSpec((F, H1 * C), lambda i: (0, 0)),
            pl.BlockSpec((1, H1 * C), lambda i: (0, 0)),
            pl.BlockSpec((H1 * C, C), lambda i: (0, 0)),
            pl.BlockSpec((1, C), lambda i: (0, 0)),
            pl.BlockSpec((1, C), lambda i: (0, 0)),
        ],
        out_specs=[
            pl.BlockSpec((blk, C), lambda i: (i, 0)),
            pl.BlockSpec((blk, 128), lambda i: (i, 0)),
            pl.BlockSpec((blk, 128), lambda i: (i, 0)),
        ],
        out_shape=[
            jax.ShapeDtypeStruct((ROWS, C), jnp.float32),
            jax.ShapeDtypeStruct((ROWS, 128), jnp.float32),
            jax.ShapeDtypeStruct((ROWS, 128), jnp.float32),
        ],
    )(agg1, W1, b1[None, :], W2, a2_src, a2_dst)


def _t3_body(p_ref, b2_ref, batch_ref, wfc_ref, bfc_ref, out_ref,
             sums_sc, cnts_sc):
    i = pl.program_id(0)
    d = p_ref[:, C:C + 1] + 1e-16
    h2 = p_ref[:, 0:C] / d + b2_ref[...]
    h2 = jnp.where(h2 > 0.0, h2, jnp.exp(h2) - 1.0)
    bb = batch_ref[0, 0, :][None, :]
    gid = lax.broadcasted_iota(jnp.int32, (G, 1), 0)
    mask = (bb == gid).astype(jnp.float32)

    @pl.when(i == 0)
    def _():
        sums_sc[...] = jnp.zeros_like(sums_sc)
        cnts_sc[...] = jnp.zeros_like(cnts_sc)

    sums_sc[...] += jnp.dot(mask, h2, preferred_element_type=jnp.float32)
    cnts_sc[...] += jnp.sum(mask, axis=1, keepdims=True)

    @pl.when(i == pl.num_programs(0) - 1)
    def _():
        pooled = sums_sc[...] / jnp.maximum(cnts_sc[...], 1.0)
        out_ref[...] = jnp.dot(pooled, wfc_ref[...],
                               preferred_element_type=jnp.float32) + bfc_ref[...]


def _t3(agg2, b2, batch, Wfc, bfc):
    nb = 10
    blk = N // nb
    return pl.pallas_call(
        _t3_body,
        grid=(nb,),
        in_specs=[
            pl.BlockSpec((blk, 256), lambda i: (i, 0)),
            pl.BlockSpec((1, C), lambda i: (0, 0)),
            pl.BlockSpec((1, 1, blk), lambda i: (i, 0, 0)),
            pl.BlockSpec((C, NCLS), lambda i: (0, 0)),
            pl.BlockSpec((1, NCLS), lambda i: (0, 0)),
        ],
        out_specs=pl.BlockSpec((G, NCLS), lambda i: (0, 0)),
        out_shape=jax.ShapeDtypeStruct((G, NCLS), jnp.float32),
        scratch_shapes=[
            pltpu.VMEM((G, C), jnp.float32),
            pltpu.VMEM((G, 1), jnp.float32),
        ],
    )(agg2, b2[None, :], batch.reshape(nb, 1, blk), Wfc, bfc[None, :])


def kernel(x, edge_index, batch, W1, a1_src, a1_dst, b1, W2, a2_src, a2_dst,
           b2, Wfc, bfc):
    src = edge_index[0]
    dst = edge_index[1]

    as1, ad1 = _t1(x, W1, a1_src, a1_dst)
    ad1p = jnp.concatenate(
        [ad1, jnp.zeros((ADP - N, 128), jnp.float32)], axis=0)

    agg1 = _sc_agg(src, dst, x, as1, ad1p, H1).reshape(ROWS, (H1 + 1) * 128)

    xp2, as2, ad2 = _t2(agg1, W1, b1, W2, a2_src, a2_dst)
    ad2p = jnp.concatenate(
        [ad2, jnp.zeros((ADP - ROWS, 128), jnp.float32)], axis=0)

    agg2 = _sc_agg(src, dst, xp2, as2, ad2p, 1).reshape(ROWS, 256)

    return _t3(agg2, b2, batch, Wfc, bfc)
